# baseline (speedup 1.0000x reference)
import jax
import jax.numpy as jnp
from jax import lax
from jax.experimental import pallas as pl
from jax.experimental.pallas import tpu as pltpu
from jax.experimental.pallas import tpu_sc as plsc

D = 64
SCALE = float(D) ** 0.5
NC, NS = 2, 16
NW = NC * NS
B_TOTAL = 4096 * 200
PER_W = B_TOTAL // NW
C = 128
NCHUNK = PER_W // C


def _embed_body(x_hbm, table_hbm, out_hbm, idx_v, rows_v, sem):
    wid = lax.axis_index("s") * NC + lax.axis_index("c")
    base = wid * PER_W

    def chunk(c, carry):
        row0 = base + c * C
        pltpu.sync_copy(x_hbm.at[pl.ds(row0, C)], idx_v)
        pltpu.async_copy(table_hbm.at[idx_v], rows_v, sem).wait()

        def srow(i, carry2):
            for j in range(D // 16):
                sl = pl.ds(j * 16, 16)
                rows_v[i, sl] = rows_v[i, sl] * SCALE
            return carry2

        lax.fori_loop(0, C, srow, 0)
        pltpu.sync_copy(rows_v, out_hbm.at[pl.ds(row0, C)])
        return carry

    lax.fori_loop(0, NCHUNK, chunk, 0)


@jax.jit
def kernel(x, table):
    xf = x.reshape(-1).astype(jnp.int32)
    mesh = plsc.VectorSubcoreMesh(
        core_axis_name="c", subcore_axis_name="s",
        num_cores=NC, num_subcores=NS,
    )
    out = pl.kernel(
        _embed_body,
        out_type=jax.ShapeDtypeStruct((B_TOTAL, D), jnp.float32),
        mesh=mesh,
        scratch_types=[
            pltpu.VMEM((C,), jnp.int32),
            pltpu.VMEM((C, D), jnp.float32),
            pltpu.SemaphoreType.DMA,
        ],
        compiler_params=pltpu.CompilerParams(use_tc_tiling_on_sc=False),
    )(xf, table)
    return out.reshape(x.shape[0], x.shape[1], D)


# speedup vs baseline: 1.2614x; 1.2614x over previous
import jax
import jax.numpy as jnp
from jax import lax
from jax.experimental import pallas as pl
from jax.experimental.pallas import tpu as pltpu
from jax.experimental.pallas import tpu_sc as plsc

D = 64
SCALE = float(D) ** 0.5
NC, NS = 2, 16
NW = NC * NS
B_TOTAL = 4096 * 200
PER_W = B_TOTAL // NW
C = 128
NBUF = 4
NCHUNK = PER_W // C
NGROUP = NCHUNK // NBUF


def _embed_body(x_hbm, table_hbm, out_hbm, idx_v, rows_v, sem_g, sem_o):
    wid = lax.axis_index("s") * NC + lax.axis_index("c")
    base = wid * PER_W

    pltpu.sync_copy(x_hbm.at[pl.ds(base, PER_W)], idx_v)

    def fire_gather(c, b):
        pltpu.async_copy(
            table_hbm.at[idx_v.at[pl.ds(c * C, C)]], rows_v.at[b], sem_g.at[b]
        )

    def wait_gather(b):
        pltpu.make_async_copy(
            table_hbm.at[pl.ds(0, C)], rows_v.at[b], sem_g.at[b]
        ).wait()

    def fire_out(c, b):
        pltpu.async_copy(
            rows_v.at[b], out_hbm.at[pl.ds(base + c * C, C)], sem_o.at[b]
        )

    def wait_out(b):
        pltpu.make_async_copy(
            rows_v.at[b], out_hbm.at[pl.ds(0, C)], sem_o.at[b]
        ).wait()

    for b in range(NBUF):
        fire_gather(b, b)

    def group(g, carry):
        for b in range(NBUF):
            c = g * NBUF + b
            wait_gather(b)

            @plsc.parallel_loop(0, C, step=1, unroll=8)
            def _scale(i):
                for j in range(D // 16):
                    sl = pl.ds(j * 16, 16)
                    rows_v[b, i, sl] = rows_v[b, i, sl] * SCALE

            fire_out(c, b)

        @pl.when(g + 1 < NGROUP)
        def _prefetch():
            for b in range(NBUF):
                wait_out(b)
                fire_gather((g + 1) * NBUF + b, b)

        return carry

    lax.fori_loop(0, NGROUP, group, 0)

    for b in range(NBUF):
        wait_out(b)


@jax.jit
def kernel(x, table):
    xf = x.reshape(-1).astype(jnp.int32)
    mesh = plsc.VectorSubcoreMesh(
        core_axis_name="c", subcore_axis_name="s",
        num_cores=NC, num_subcores=NS,
    )
    out = pl.kernel(
        _embed_body,
        out_type=jax.ShapeDtypeStruct((B_TOTAL, D), jnp.float32),
        mesh=mesh,
        scratch_types=[
            pltpu.VMEM((PER_W,), jnp.int32),
            pltpu.VMEM((NBUF, C, D), jnp.float32),
            pltpu.SemaphoreType.DMA((NBUF,)),
            pltpu.SemaphoreType.DMA((NBUF,)),
        ],
        compiler_params=pltpu.CompilerParams(use_tc_tiling_on_sc=False),
    )(xf, table)
    return out.reshape(x.shape[0], x.shape[1], D)


# speedup vs baseline: 1.5557x; 1.2333x over previous
import jax
import jax.numpy as jnp
from jax import lax
from jax.experimental import pallas as pl
from jax.experimental.pallas import tpu as pltpu
from jax.experimental.pallas import tpu_sc as plsc

D = 64
DP = 128
SCALE = float(D) ** 0.5
NC, NS = 2, 16
NW = NC * NS
B_TOTAL = 4096 * 200
PER_W = B_TOTAL // NW
C = 128
NBUF = 2
NCHUNK = PER_W // C
NGROUP = NCHUNK // NBUF


def _embed_body(x_hbm, tab_hbm, out_hbm, idx_v, rowsP, rows64, sem_g, sem_o):
    wid = lax.axis_index("s") * NC + lax.axis_index("c")
    base = wid * PER_W

    pltpu.sync_copy(x_hbm.at[pl.ds(base, PER_W)], idx_v)

    def fire_gather(c, b):
        pltpu.async_copy(
            tab_hbm.at[idx_v.at[pl.ds(c * C, C)]], rowsP.at[b], sem_g.at[b]
        )

    def wait_gather(b):
        pltpu.make_async_copy(
            tab_hbm.at[pl.ds(0, C)], rowsP.at[b], sem_g.at[b]
        ).wait()

    def fire_out(c, b):
        pltpu.async_copy(
            rows64.at[b], out_hbm.at[pl.ds(base + c * C, C)], sem_o.at[b]
        )

    def wait_out(b):
        pltpu.make_async_copy(
            rows64.at[b], out_hbm.at[pl.ds(0, C)], sem_o.at[b]
        ).wait()

    for b in range(NBUF):
        fire_gather(b, b)

    def group(g, carry):
        for b in range(NBUF):
            c = g * NBUF + b
            wait_gather(b)

            @pl.when(g > 0)
            def _():
                wait_out(b)

            @plsc.parallel_loop(0, C, step=1, unroll=8)
            def _scale(i):
                for j in range(D // 16):
                    sl = pl.ds(j * 16, 16)
                    rows64[b, i, sl] = rowsP[b, i, sl] * SCALE

            fire_out(c, b)

            @pl.when(g + 1 < NGROUP)
            def _():
                fire_gather(c + NBUF, b)

        return carry

    lax.fori_loop(0, NGROUP, group, 0)

    for b in range(NBUF):
        wait_out(b)


@jax.jit
def kernel(x, table):
    xf = x.reshape(-1).astype(jnp.int32)
    tabP = jnp.pad(table, ((0, 0), (0, DP - D)))
    mesh = plsc.VectorSubcoreMesh(
        core_axis_name="c", subcore_axis_name="s",
        num_cores=NC, num_subcores=NS,
    )
    out = pl.kernel(
        _embed_body,
        out_type=jax.ShapeDtypeStruct((B_TOTAL, D), jnp.float32),
        mesh=mesh,
        scratch_types=[
            pltpu.VMEM((PER_W,), jnp.int32),
            pltpu.VMEM((NBUF, C, DP), jnp.float32),
            pltpu.VMEM((NBUF, C, D), jnp.float32),
            pltpu.SemaphoreType.DMA((NBUF,)),
            pltpu.SemaphoreType.DMA((NBUF,)),
        ],
        compiler_params=pltpu.CompilerParams(use_tc_tiling_on_sc=True),
    )(xf, tabP)
    return out.reshape(x.shape[0], x.shape[1], D)
